# baseline (speedup 1.0000x reference)
import jax
import jax.numpy as jnp
from jax import lax
from jax.experimental import pallas as pl
from jax.experimental.pallas import tpu as pltpu

_BLK = 2048
_BLK2 = 2048


def _knn_body(dxyz_ref, sxyzt_ref, ddata_ref, sdata_ref, w_ref,
              yraw_ref, sum_ref, sq_ref, ps_ref):
    ns = sxyzt_ref.shape[1]
    c = w_ref.shape[0]
    n = pl.program_id(1)

    @pl.when(n == 0)
    def _project_sparse():
        ps_ref[:c, :] = jnp.dot(w_ref[:, c:], sdata_ref[0],
                                preferred_element_type=jnp.float32)
        ps_ref[c:, :] = jnp.ones_like(ps_ref[c:, :])

    dxyz = dxyz_ref[0]
    sxyzt = sxyzt_ref[0]
    snorm = jnp.sum(sxyzt * sxyzt, axis=1, keepdims=True)
    dnorm = jnp.sum(dxyz * dxyz, axis=0, keepdims=True)
    dot = jnp.dot(sxyzt, dxyz * -2.0, preferred_element_type=jnp.float32)
    t = snorm + dot

    a = jnp.float32(0.0)
    for k in range(3):
        v = jnp.min(t, axis=0, keepdims=True)
        wk = 1.0 / (jnp.maximum(v + dnorm, 0.0) + 1e-8)
        m = t <= v
        a = jnp.where(m, wk, a)
        if k < 2:
            t = jnp.where(m, jnp.inf, t)

    interp_ext = jnp.dot(ps_ref[...], a, preferred_element_type=jnp.float32)
    recip = 1.0 / interp_ext[c:c + 1, :]
    y = (jnp.dot(w_ref[:, :c], ddata_ref[0], preferred_element_type=jnp.float32)
         + interp_ext[:c, :] * recip)
    yraw_ref[0] = y

    @pl.when((pl.program_id(0) == 0) & (n == 0))
    def _init_stats():
        sum_ref[...] = jnp.zeros_like(sum_ref)
        sq_ref[...] = jnp.zeros_like(sq_ref)

    sum_ref[...] += jnp.sum(y, axis=1, keepdims=True)
    sq_ref[...] += jnp.sum(y * y, axis=1, keepdims=True)


def _bn_body(yraw_ref, sum_ref, sq_ref, gamma_ref, beta_ref, total_ref,
             out_ref):
    inv_n = 1.0 / total_ref[0]
    mean = sum_ref[...] * inv_n
    var = sq_ref[...] * inv_n - mean * mean
    scale = gamma_ref[...] * lax.rsqrt(var + 1e-5)
    shift = beta_ref[...] - mean * scale
    z = yraw_ref[0] * scale + shift
    out_ref[0] = jnp.where(z > 0, z, 0.2 * z)


def kernel(dense_points_xyz, sparse_points_xyz, dense_points_data,
           sparse_points_data, W, gamma, beta):
    b, _, nd = dense_points_xyz.shape
    ns = sparse_points_xyz.shape[2]
    c = W.shape[0]

    sxyz_t = sparse_points_xyz.transpose(0, 2, 1)

    yraw, ysum, ysq = pl.pallas_call(
        _knn_body,
        grid=(b, nd // _BLK),
        in_specs=[
            pl.BlockSpec((1, 3, _BLK), lambda i, j: (i, 0, j)),
            pl.BlockSpec((1, ns, 3), lambda i, j: (i, 0, 0)),
            pl.BlockSpec((1, c, _BLK), lambda i, j: (i, 0, j)),
            pl.BlockSpec((1, 2 * c, ns), lambda i, j: (i, 0, 0)),
            pl.BlockSpec((c, 3 * c), lambda i, j: (0, 0)),
        ],
        out_specs=[
            pl.BlockSpec((1, c, _BLK), lambda i, j: (i, 0, j)),
            pl.BlockSpec((c, 1), lambda i, j: (0, 0)),
            pl.BlockSpec((c, 1), lambda i, j: (0, 0)),
        ],
        out_shape=[
            jax.ShapeDtypeStruct((b, c, nd), jnp.float32),
            jax.ShapeDtypeStruct((c, 1), jnp.float32),
            jax.ShapeDtypeStruct((c, 1), jnp.float32),
        ],
        scratch_shapes=[pltpu.VMEM((c + 8, ns), jnp.float32)],
    )(dense_points_xyz, sxyz_t, dense_points_data, sparse_points_data, W)

    total = jnp.full((1,), float(b * nd), jnp.float32)
    ybn = pl.pallas_call(
        _bn_body,
        grid=(b, nd // _BLK2),
        in_specs=[
            pl.BlockSpec((1, c, _BLK2), lambda i, j: (i, 0, j)),
            pl.BlockSpec((c, 1), lambda i, j: (0, 0)),
            pl.BlockSpec((c, 1), lambda i, j: (0, 0)),
            pl.BlockSpec((c, 1), lambda i, j: (0, 0)),
            pl.BlockSpec((c, 1), lambda i, j: (0, 0)),
            pl.BlockSpec(memory_space=pltpu.SMEM),
        ],
        out_specs=pl.BlockSpec((1, c, _BLK2), lambda i, j: (i, 0, j)),
        out_shape=jax.ShapeDtypeStruct((b, c, nd), jnp.float32),
    )(yraw, ysum, ysq, gamma.reshape(c, 1), beta.reshape(c, 1), total)

    return (ybn, dense_points_xyz)
